# baseline (speedup 1.0000x reference)
import jax
import jax.numpy as jnp
from jax.experimental import pallas as pl
from jax.experimental.pallas import tpu as pltpu

N_NODES = 10000
NFEAT = 128
NHID = 32
N_CLUSTERS = 10
ALPHA = 0.2
CH = 400
NCH = N_NODES // CH
NBUF = 3


def _in_copy(adj_hbm, buf, sem, chunk, slot):
    return pltpu.make_async_copy(
        adj_hbm.at[pl.ds(chunk * CH, CH), :], buf.at[slot], sem.at[slot]
    )


def _out_copies(ostg, qstg, out_hbm, q_hbm, osem, qsem, chunk, oslot):
    rows = pl.ds(chunk * CH, CH)
    return (
        pltpu.make_async_copy(ostg.at[oslot], out_hbm.at[rows, :], osem.at[oslot]),
        pltpu.make_async_copy(qstg.at[oslot], q_hbm.at[rows, :], qsem.at[oslot]),
    )


def _gcdec_body(w_ref, b_ref, mu_ref, x_hbm, adj_hbm, out_hbm, q_hbm,
                buf, xv, support, ostg, qstg, sem, xsem, osem, qsem):
    x_copy = pltpu.make_async_copy(x_hbm, xv, xsem)
    x_copy.start()
    for k in range(NBUF):
        _in_copy(adj_hbm, buf, sem, k, k).start()
    x_copy.wait()

    support[:] = jnp.dot(xv[:], w_ref[:], preferred_element_type=jnp.float32)
    mu = mu_ref[:]
    mu_sq = jnp.sum(mu * mu, axis=1, keepdims=True).reshape(1, N_CLUSTERS)

    def step(i, carry):
        slot = jax.lax.rem(i, NBUF)
        oslot = jax.lax.rem(i, 2)
        _in_copy(adj_hbm, buf, sem, i, slot).wait()
        out_blk = (
            jnp.dot(buf[slot], support[:], preferred_element_type=jnp.float32)
            + b_ref[:]
        )

        @pl.when(i + NBUF < NCH)
        def _():
            _in_copy(adj_hbm, buf, sem, i + NBUF, slot).start()

        cross = jax.lax.dot_general(
            out_blk, mu, (((1,), (1,)), ((), ())),
            preferred_element_type=jnp.float32,
        )
        d2 = (
            jnp.sum(out_blk * out_blk, axis=1, keepdims=True) + mu_sq
            - 2.0 * cross
        )
        q = 1.0 / (1.0 + d2 / ALPHA + 1e-08)
        q = q ** (ALPHA + 1.0) / 2.0
        q = q / jnp.sum(q, axis=1, keepdims=True)

        @pl.when(i >= 2)
        def _():
            oc, qc = _out_copies(
                ostg, qstg, out_hbm, q_hbm, osem, qsem, i - 2, oslot
            )
            oc.wait()
            qc.wait()

        ostg[oslot] = out_blk
        qstg[oslot] = q
        oc, qc = _out_copies(ostg, qstg, out_hbm, q_hbm, osem, qsem, i, oslot)
        oc.start()
        qc.start()
        return carry

    jax.lax.fori_loop(0, NCH, step, 0)

    for t in (NCH - 2, NCH - 1):
        oc, qc = _out_copies(
            ostg, qstg, out_hbm, q_hbm, osem, qsem, t, t % 2
        )
        oc.wait()
        qc.wait()


def kernel(x, adj, W, b, mu):
    b2 = b.reshape(1, NHID)
    out, q = pl.pallas_call(
        _gcdec_body,
        in_specs=[
            pl.BlockSpec((NFEAT, NHID), lambda: (0, 0)),
            pl.BlockSpec((1, NHID), lambda: (0, 0)),
            pl.BlockSpec((N_CLUSTERS, NHID), lambda: (0, 0)),
            pl.BlockSpec(memory_space=pltpu.MemorySpace.HBM),
            pl.BlockSpec(memory_space=pltpu.MemorySpace.HBM),
        ],
        out_specs=[
            pl.BlockSpec(memory_space=pltpu.MemorySpace.HBM),
            pl.BlockSpec(memory_space=pltpu.MemorySpace.HBM),
        ],
        out_shape=[
            jax.ShapeDtypeStruct((N_NODES, NHID), jnp.float32),
            jax.ShapeDtypeStruct((N_NODES, N_CLUSTERS), jnp.float32),
        ],
        scratch_shapes=[
            pltpu.VMEM((NBUF, CH, N_NODES), jnp.float32),
            pltpu.VMEM((N_NODES, NFEAT), jnp.float32),
            pltpu.VMEM((N_NODES, NHID), jnp.float32),
            pltpu.VMEM((2, CH, NHID), jnp.float32),
            pltpu.VMEM((2, CH, N_CLUSTERS), jnp.float32),
            pltpu.SemaphoreType.DMA((NBUF,)),
            pltpu.SemaphoreType.DMA,
            pltpu.SemaphoreType.DMA((2,)),
            pltpu.SemaphoreType.DMA((2,)),
        ],
        compiler_params=pltpu.CompilerParams(
            vmem_limit_bytes=64 * 1024 * 1024,
        ),
    )(W, b2, mu, x, adj)
    return (out, q)


# speedup vs baseline: 1.0215x; 1.0215x over previous
import jax
import jax.numpy as jnp
from jax.experimental import pallas as pl
from jax.experimental.pallas import tpu as pltpu

N_NODES = 10000
NFEAT = 128
NHID = 32
N_CLUSTERS = 10
ALPHA = 0.2
BM = 240
GRID = -(-N_NODES // BM)


def _gcdec_body(x_ref, adj_ref, w_ref, b_ref, mu_ref, out_ref, q_ref, support_ref):
    i = pl.program_id(0)

    @pl.when(i == 0)
    def _():
        support_ref[:] = jnp.dot(
            x_ref[:], w_ref[:], preferred_element_type=jnp.float32
        )

    out_blk = (
        jnp.dot(adj_ref[:], support_ref[:], preferred_element_type=jnp.float32)
        + b_ref[:]
    )
    out_ref[:] = out_blk

    mu = mu_ref[:]
    cross = jax.lax.dot_general(
        out_blk, mu, (((1,), (1,)), ((), ())),
        preferred_element_type=jnp.float32,
    )
    d2 = (
        jnp.sum(out_blk * out_blk, axis=1, keepdims=True)
        + jnp.sum(mu * mu, axis=1, keepdims=True).reshape(1, N_CLUSTERS)
        - 2.0 * cross
    )
    t = 1.0 + d2 * (1.0 / ALPHA)
    q = jnp.exp((-(ALPHA + 1.0)) * jnp.log(t))
    q_ref[:] = q / jnp.sum(q, axis=1, keepdims=True)


def kernel(x, adj, W, b, mu):
    b2 = b.reshape(1, NHID)
    out, q = pl.pallas_call(
        _gcdec_body,
        grid=(GRID,),
        in_specs=[
            pl.BlockSpec((N_NODES, NFEAT), lambda i: (0, 0)),
            pl.BlockSpec((BM, N_NODES), lambda i: (i, 0)),
            pl.BlockSpec((NFEAT, NHID), lambda i: (0, 0)),
            pl.BlockSpec((1, NHID), lambda i: (0, 0)),
            pl.BlockSpec((N_CLUSTERS, NHID), lambda i: (0, 0)),
        ],
        out_specs=[
            pl.BlockSpec((BM, NHID), lambda i: (i, 0)),
            pl.BlockSpec((BM, N_CLUSTERS), lambda i: (i, 0)),
        ],
        out_shape=[
            jax.ShapeDtypeStruct((N_NODES, NHID), jnp.float32),
            jax.ShapeDtypeStruct((N_NODES, N_CLUSTERS), jnp.float32),
        ],
        scratch_shapes=[pltpu.VMEM((N_NODES, NHID), jnp.float32)],
        compiler_params=pltpu.CompilerParams(
            vmem_limit_bytes=64 * 1024 * 1024,
        ),
    )(x, adj, W, b2, mu)
    return (out, q)
